# Initial kernel scaffold; baseline (speedup 1.0000x reference)
import jax
import jax.numpy as jnp
from jax.experimental import pallas as pl


def kernel(states, actions, W_embed, b_embed, dnd_keys, dnd_values):
    raise NotImplementedError("write your pallas kernel here")


# speedup vs baseline: 12.9285x; 12.9285x over previous
import functools
from typing import Any

import jax
import jax.numpy as jnp
from jax.experimental import pallas as pl
from jax.experimental.pallas import tpu as pltpu

TOP_P = 50
DELTA = 1e-3
_LANE = 128


def _round_up(x: int, m: int) -> int:
    return (x + m - 1) // m * m


def _nec_body(nq, nc, cdim, mem, mem_pad, n_actions, top_p,
              states_ref, actions_ref, w_ref, b_ref, kt_ref, vals_ref,
              out_ref, keys_scr, d2_scr):
    q = pl.program_id(0)
    c = pl.program_id(1)
    bq = states_ref.shape[0]

    @pl.when(c == 0)
    def _compute_keys():
        st = states_ref[...]
        w = w_ref[...]
        b = b_ref[0:1, 0:w.shape[1]]
        keys_scr[...] = jnp.dot(st, w, preferred_element_type=jnp.float32) + b

    keys = keys_scr[...]
    qn = jnp.sum(keys * keys, axis=1, keepdims=True)
    af = actions_ref[:, 0:1]

    d2_by_a = []
    for a in range(n_actions):
        kt = kt_ref[a]
        qk = jnp.dot(keys, kt, preferred_element_type=jnp.float32)
        kn = jnp.sum(kt * kt, axis=0, keepdims=True)
        d2_by_a.append(qn - 2.0 * qk + kn)
    if n_actions == 4:
        d2_sel = jnp.where(
            af < 1.5,
            jnp.where(af < 0.5, d2_by_a[0], d2_by_a[1]),
            jnp.where(af < 2.5, d2_by_a[2], d2_by_a[3]))
    else:
        d2_sel = d2_by_a[0]
        for a in range(1, n_actions):
            d2_sel = jnp.where(af == float(a), d2_by_a[a], d2_sel)

    lane_idx = jax.lax.broadcasted_iota(jnp.int32, d2_sel.shape, 1) + c * cdim
    d2_sel = jnp.where(lane_idx >= mem, jnp.inf, d2_sel)
    d2_scr[:, pl.ds(c * cdim, cdim)] = d2_sel

    @pl.when(c == nc - 1)
    def _select_and_reduce():
        d2 = d2_scr[...]
        di = jax.lax.bitcast_convert_type(jnp.maximum(d2, 0.0), jnp.int32)

        def bis_body(_, lohi):
            lo, hi = lohi
            mid = lo + jax.lax.shift_right_logical(hi - lo, 1)
            cnt = jnp.sum((di <= mid).astype(jnp.int32), axis=1,
                          keepdims=True)
            ge = cnt >= top_p
            return (jnp.where(ge, lo, mid + 1), jnp.where(ge, mid, hi))

        lo0 = jnp.zeros((bq, 1), jnp.int32)
        hi0 = jnp.full((bq, 1), jnp.int32(0x7F800000))
        _, t_int = jax.lax.fori_loop(0, 31, bis_body, (lo0, hi0))
        t = jax.lax.bitcast_convert_type(t_int, jnp.float32)

        w = 1.0 / (d2 + DELTA)
        mask_lt = d2 < t
        mask_eq = d2 == t
        if n_actions == 4:
            v_sel = jnp.where(
                af < 1.5,
                jnp.where(af < 0.5, vals_ref[0:1, :], vals_ref[1:2, :]),
                jnp.where(af < 2.5, vals_ref[2:3, :], vals_ref[3:4, :]))
        else:
            v_sel = jnp.broadcast_to(vals_ref[0:1, :], d2.shape)
            for a in range(1, n_actions):
                v_sel = jnp.where(af == float(a), vals_ref[a:a + 1, :], v_sel)

        s_w = jnp.sum(jnp.where(mask_lt, w, 0.0), axis=1, keepdims=True)
        s_wv = jnp.sum(jnp.where(mask_lt, w * v_sel, 0.0), axis=1,
                       keepdims=True)
        c_lt = jnp.sum(mask_lt.astype(jnp.int32), axis=1, keepdims=True)
        needed = top_p - c_lt

        idx = jax.lax.broadcasted_iota(jnp.int32, d2.shape, 1)

        def jbis_body(_, lohi):
            lo, hi = lohi
            mid = lo + jax.lax.shift_right_logical(hi - lo, 1)
            cnt = jnp.sum((mask_eq & (idx <= mid)).astype(jnp.int32), axis=1,
                          keepdims=True)
            ge = cnt >= needed
            return (jnp.where(ge, lo, mid + 1), jnp.where(ge, mid, hi))

        nbits = max(1, (mem_pad - 1).bit_length())
        lo0j = jnp.zeros((bq, 1), jnp.int32)
        hi0j = jnp.full((bq, 1), jnp.int32(mem_pad - 1))
        _, j_int = jax.lax.fori_loop(0, nbits, jbis_body, (lo0j, hi0j))

        take_eq = mask_eq & (idx <= j_int)
        wt = 1.0 / (t + DELTA)
        sv_eq = jnp.sum(jnp.where(take_eq, v_sel, 0.0), axis=1, keepdims=True)
        total_w = s_w + wt * needed.astype(jnp.float32)
        total_wv = s_wv + wt * sv_eq
        res = total_wv / total_w
        out_ref[...] = jnp.broadcast_to(res, out_ref.shape)


def kernel(states, actions, W_embed, b_embed, dnd_keys, dnd_values):
    b, d_state = states.shape
    n_actions, mem, d_key = dnd_keys.shape
    top_p = TOP_P

    nq = 8 if b % 8 == 0 else 1
    bq = b // nq
    mem_pad = _round_up(mem, 8 * _LANE)
    nc = 8
    while mem_pad % (nc * _LANE) != 0:
        nc //= 2
    cdim = mem_pad // nc

    kt = jnp.transpose(dnd_keys, (0, 2, 1))
    kt = jnp.pad(kt, ((0, 0), (0, 0), (0, mem_pad - mem)))
    vals = jnp.pad(dnd_values, ((0, 0), (0, mem_pad - mem)))
    af = jnp.broadcast_to(actions.astype(jnp.float32)[:, None], (b, _LANE))
    bpad = jnp.zeros((8, _LANE), jnp.float32).at[0, :d_key].set(b_embed)

    body = functools.partial(_nec_body, nq, nc, cdim, mem, mem_pad,
                             n_actions, top_p)
    out = pl.pallas_call(
        body,
        grid=(nq, nc),
        in_specs=[
            pl.BlockSpec((bq, d_state), lambda q, c: (q, 0)),
            pl.BlockSpec((bq, _LANE), lambda q, c: (q, 0)),
            pl.BlockSpec((d_state, d_key), lambda q, c: (0, 0)),
            pl.BlockSpec((8, _LANE), lambda q, c: (0, 0)),
            pl.BlockSpec((n_actions, d_key, cdim),
                         lambda q, c: (0, 0, c)),
            pl.BlockSpec((n_actions, mem_pad), lambda q, c: (0, 0)),
        ],
        out_specs=pl.BlockSpec((bq, _LANE), lambda q, c: (q, 0)),
        out_shape=jax.ShapeDtypeStruct((b, _LANE), jnp.float32),
        scratch_shapes=[
            pltpu.VMEM((bq, d_key), jnp.float32),
            pltpu.VMEM((bq, mem_pad), jnp.float32),
        ],
        compiler_params=pltpu.CompilerParams(
            dimension_semantics=("arbitrary", "arbitrary"),
        ),
    )(states, af, W_embed, bpad, kt, vals)
    return out[:, 0]


# speedup vs baseline: 26.1857x; 2.0254x over previous
import functools

import jax
import jax.numpy as jnp
from jax import lax
from jax.experimental import pallas as pl
from jax.experimental.pallas import tpu as pltpu
from jax.experimental.pallas import tpu_sc as plsc

TOP_P = 50
DELTA = 1e-3
_LANE = 128
_WIN = 128
_CW_CAP = 128
_EL_CAP = 256
_L = 16


def _round_up(x: int, m: int) -> int:
    return (x + m - 1) // m * m


def _tc_body(nq, nc, cdim, mem, mem_pad, n_actions, top_p, nwin,
             states_ref, actions_ref, w_ref, b_ref, kt_ref,
             d2_ref, winmin_ref, tau_ref, keys_scr, wmin_scr):
    c = pl.program_id(1)
    bq = states_ref.shape[0]
    wpc = cdim // _WIN

    @pl.when(c == 0)
    def _compute_keys():
        st = states_ref[...]
        w = w_ref[...]
        bvec = b_ref[0:1, 0:w.shape[1]]
        keys_scr[...] = jnp.dot(st, w, preferred_element_type=jnp.float32) + bvec
        wmin_scr[...] = jnp.full(wmin_scr.shape, jnp.inf, jnp.float32)

    keys = keys_scr[...]
    qn = jnp.sum(keys * keys, axis=1, keepdims=True)
    af = actions_ref[:, 0:1]

    d2_by_a = []
    for a in range(n_actions):
        kt = kt_ref[a]
        qk = jnp.dot(keys, kt, preferred_element_type=jnp.float32)
        kn = jnp.sum(kt * kt, axis=0, keepdims=True)
        d2_by_a.append(qn - 2.0 * qk + kn)
    if n_actions == 4:
        d2_sel = jnp.where(
            af < 1.5,
            jnp.where(af < 0.5, d2_by_a[0], d2_by_a[1]),
            jnp.where(af < 2.5, d2_by_a[2], d2_by_a[3]))
    else:
        d2_sel = d2_by_a[0]
        for a in range(1, n_actions):
            d2_sel = jnp.where(af == float(a), d2_by_a[a], d2_sel)

    lane_idx = jax.lax.broadcasted_iota(jnp.int32, d2_sel.shape, 1) + c * cdim
    d2_sel = jnp.where(lane_idx >= mem, jnp.inf, d2_sel)
    d2_ref[...] = d2_sel

    wmin = jnp.min(d2_sel.reshape(bq, wpc, _WIN), axis=2)
    wpad = jnp.full((bq, _LANE - wpc), jnp.inf, jnp.float32)
    wmin_scr[:, pl.ds(c * _LANE, _LANE)] = jnp.concatenate([wmin, wpad],
                                                           axis=1)

    @pl.when(c == nc - 1)
    def _tau():
        wm = wmin_scr[...]
        wi = jax.lax.bitcast_convert_type(jnp.maximum(wm, 0.0), jnp.int32)
        kth = top_p + 1

        def bis_body(_, lohi):
            lo, hi = lohi
            mid = lo + jax.lax.shift_right_logical(hi - lo, 1)
            cnt = jnp.sum((wi <= mid).astype(jnp.int32), axis=1, keepdims=True)
            ge = cnt >= kth
            return (jnp.where(ge, lo, mid + 1), jnp.where(ge, mid, hi))

        lo0 = jnp.zeros((bq, 1), jnp.int32)
        hi0 = jnp.full((bq, 1), jnp.int32(0x7F800000))
        _, t_int = jax.lax.fori_loop(0, 31, bis_body, (lo0, hi0))
        tau = jax.lax.bitcast_convert_type(t_int, jnp.float32)
        winmin_ref[...] = wm
        tau_ref[...] = jnp.broadcast_to(tau, tau_ref.shape)


def _tc_stage(states, af, W_embed, bpad, kt, n_actions, mem, mem_pad):
    b, d_state = states.shape
    d_key = W_embed.shape[1]
    nq = 8 if b % 8 == 0 else 1
    bq = b // nq
    nc = 8
    while mem_pad % (nc * _LANE) != 0:
        nc //= 2
    cdim = mem_pad // nc
    nwin = mem_pad // _WIN
    nwin_pad = nc * _LANE

    body = functools.partial(_tc_body, nq, nc, cdim, mem, mem_pad,
                             n_actions, TOP_P, nwin)
    return pl.pallas_call(
        body,
        grid=(nq, nc),
        in_specs=[
            pl.BlockSpec((bq, d_state), lambda q, c: (q, 0)),
            pl.BlockSpec((bq, _LANE), lambda q, c: (q, 0)),
            pl.BlockSpec((d_state, d_key), lambda q, c: (0, 0)),
            pl.BlockSpec((8, _LANE), lambda q, c: (0, 0)),
            pl.BlockSpec((n_actions, d_key, cdim), lambda q, c: (0, 0, c)),
        ],
        out_specs=[
            pl.BlockSpec((bq, cdim), lambda q, c: (q, c)),
            pl.BlockSpec((bq, nwin_pad), lambda q, c: (q, 0)),
            pl.BlockSpec((bq, _LANE), lambda q, c: (q, 0)),
        ],
        out_shape=[
            jax.ShapeDtypeStruct((b, mem_pad), jnp.float32),
            jax.ShapeDtypeStruct((b, nwin_pad), jnp.float32),
            jax.ShapeDtypeStruct((b, _LANE), jnp.float32),
        ],
        scratch_shapes=[
            pltpu.VMEM((bq, d_key), jnp.float32),
            pltpu.VMEM((bq, nwin_pad), jnp.float32),
        ],
        compiler_params=pltpu.CompilerParams(
            dimension_semantics=("arbitrary", "arbitrary"),
        ),
    )(states, af, W_embed, bpad, kt)


def _lane_scalar_f32(vec, lane):
    io = lax.iota(jnp.int32, _L)
    return jnp.min(jnp.where(io == lane, vec, jnp.float32(jnp.inf)), axis=0)


def _lane_scalar_i32(vec, lane):
    io = lax.iota(jnp.int32, _L)
    return jnp.max(jnp.where(io == lane, vec, jnp.int32(0)), axis=0)


def _sc_stage(d2_3d, winmin, tau1d, vals3d, actions, b, nwin, wpc, top_p):
    nwin_pad = winmin.shape[1]
    mesh = plsc.VectorSubcoreMesh(core_axis_name="c", subcore_axis_name="s")
    try:
        info = plsc.get_sparse_core_info()
        nc_, ns_ = info.num_cores, info.num_subcores
    except ValueError:
        nc_, ns_ = 2, 16
    nw_workers = nc_ * ns_
    rows_per_w = b // nw_workers
    cap = _CW_CAP
    ecap = _EL_CAP
    nvec = ecap // _L
    pad_win = nwin - 1
    n_chunks = nwin_pad // _L

    @functools.partial(
        pl.kernel,
        out_type=jax.ShapeDtypeStruct((b,), jnp.float32),
        mesh=mesh,
        scratch_types=[
            pltpu.VMEM((nwin_pad,), jnp.float32),
            pltpu.VMEM((cap,), jnp.int32),
            pltpu.VMEM((cap, _WIN), jnp.float32),
            pltpu.VMEM((cap, _WIN), jnp.float32),
            pltpu.VMEM((ecap,), jnp.float32),
            pltpu.VMEM((ecap,), jnp.float32),
            pltpu.VMEM((ecap,), jnp.int32),
            pltpu.VMEM((_L,), jnp.float32),
            pltpu.VMEM((_L,), jnp.int32),
            pltpu.VMEM((_L,), jnp.float32),
            pltpu.SemaphoreType.DMA,
            pltpu.SemaphoreType.DMA,
        ],
        compiler_params=pltpu.CompilerParams(needs_layout_passes=False),
    )
    def sc_kernel(d2_hbm, winmin_hbm, tau_hbm, vals_hbm, act_hbm, out_hbm,
                  wm_v, cw_v, d2w_v, vw_v, cd_v, cv_v, cg_v,
                  tau_v, act_v, out_v, sem1, sem2):
        wid = lax.axis_index("s") * nc_ + lax.axis_index("c")
        base = wid * rows_per_w
        pltpu.sync_copy(tau_hbm.at[pl.ds(base, rows_per_w)], tau_v)
        pltpu.sync_copy(act_hbm.at[pl.ds(base, rows_per_w)], act_v)

        for j in range(cap // _L):
            cw_v[pl.ds(j * _L, _L)] = lax.iota(jnp.int32, _L) + j * _L

        def do_row(r, _carry):
            row = base + r
            pltpu.sync_copy(winmin_hbm.at[row], wm_v)
            tau_r = _lane_scalar_f32(tau_v[...], r)
            a_r = _lane_scalar_i32(act_v[...], r)

            def scan_chunk(j, cursor):
                wm = wm_v[pl.ds(j * _L, _L)]
                msk = wm <= tau_r
                pos = cursor + jnp.cumsum(msk.astype(jnp.int32)) - 1
                pos = jnp.minimum(pos, cap - 1)
                p = lax.iota(jnp.int32, _L) + j * _L
                idxs = lax.shift_right_logical(p, 7) * wpc + (p & (_LANE - 1))
                plsc.store_scatter(cw_v, [pos], idxs, mask=msk)
                return cursor + plsc.all_reduce_population_count(msk)

            nw_cnt = lax.fori_loop(0, n_chunks, scan_chunk,
                                   jnp.zeros((_L,), jnp.int32))

            cp1 = pltpu.async_copy(d2_hbm.at[row].at[cw_v], d2w_v, sem1)
            cp2 = pltpu.async_copy(vals_hbm.at[a_r].at[cw_v], vw_v, sem2)
            cp1.wait()
            cp2.wait()

            for j in range(nvec):
                cd_v[pl.ds(j * _L, _L)] = jnp.full((_L,), jnp.inf, jnp.float32)

            nwindows = jnp.max(jnp.minimum(nw_cnt, cap), axis=0)

            def compact_win(j, cursor):
                base16 = (j // _L) * _L
                wchunk = cw_v[pl.ds(base16, _L)]
                lane = j - base16
                wsel = _lane_scalar_i32(wchunk, lane)
                for h in range(_WIN // _L):
                    d = d2w_v[j, pl.ds(h * _L, _L)]
                    v = vw_v[j, pl.ds(h * _L, _L)]
                    msk = d <= tau_r
                    pos = cursor + jnp.cumsum(msk.astype(jnp.int32)) - 1
                    pos = jnp.minimum(pos, ecap - 1)
                    gidx = wsel * _WIN + h * _L + lax.iota(jnp.int32, _L)
                    plsc.store_scatter(cd_v, [pos], d, mask=msk)
                    plsc.store_scatter(cv_v, [pos], v, mask=msk)
                    plsc.store_scatter(cg_v, [pos], gidx, mask=msk)
                    cursor = cursor + plsc.all_reduce_population_count(msk)
                return cursor

            lax.fori_loop(0, nwindows, compact_win,
                          jnp.zeros((_L,), jnp.int32))

            kth = jnp.full((_L,), top_p, jnp.int32)

            def bis_body(_, lohi):
                lo, hi = lohi
                mid = lo + lax.shift_right_logical(hi - lo, 1)
                cnt = jnp.zeros((_L,), jnp.int32)
                for j in range(nvec):
                    d = cd_v[pl.ds(j * _L, _L)]
                    di = plsc.bitcast(jnp.maximum(d, 0.0), jnp.int32)
                    cnt = cnt + plsc.all_reduce_population_count(di <= mid)
                ge = cnt >= kth
                return (jnp.where(ge, lo, mid + 1), jnp.where(ge, mid, hi))

            lo0 = jnp.zeros((_L,), jnp.int32)
            hi0 = jnp.full((_L,), 0x7F800000, jnp.int32)
            _, t_int = lax.fori_loop(0, 31, bis_body, (lo0, hi0))
            t_f = plsc.bitcast(t_int, jnp.float32)

            s_w = jnp.zeros((_L,), jnp.float32)
            s_wv = jnp.zeros((_L,), jnp.float32)
            c_lt = jnp.zeros((_L,), jnp.int32)
            c_eq = jnp.zeros((_L,), jnp.int32)
            for j in range(nvec):
                d = cd_v[pl.ds(j * _L, _L)]
                v = cv_v[pl.ds(j * _L, _L)]
                di = plsc.bitcast(jnp.maximum(d, 0.0), jnp.int32)
                m_lt = di < t_int
                m_eq = di == t_int
                w = 1.0 / (d + DELTA)
                s_w = s_w + jnp.where(m_lt, w, 0.0)
                s_wv = s_wv + jnp.where(m_lt, w * v, 0.0)
                c_lt = c_lt + plsc.all_reduce_population_count(m_lt)
                c_eq = c_eq + plsc.all_reduce_population_count(m_eq)

            needed = jnp.full((_L,), top_p, jnp.int32) - c_lt
            need_sc = _lane_scalar_i32(needed, 0)
            ceq_sc = _lane_scalar_i32(c_eq, 0)

            def eq_sum(limit_v):
                acc = jnp.zeros((_L,), jnp.float32)
                for j in range(nvec):
                    d = cd_v[pl.ds(j * _L, _L)]
                    v = cv_v[pl.ds(j * _L, _L)]
                    gi = cg_v[pl.ds(j * _L, _L)]
                    di = plsc.bitcast(jnp.maximum(d, 0.0), jnp.int32)
                    m = (di == t_int) & (gi <= limit_v)
                    acc = acc + jnp.where(m, v, 0.0)
                return acc

            def tie_fast(_):
                return eq_sum(jnp.full((_L,), 2147483647, jnp.int32))

            def tie_slow(_):
                def jbis_body(_i, lohi):
                    lo, hi = lohi
                    mid = lo + lax.shift_right_logical(hi - lo, 1)
                    cnt = jnp.zeros((_L,), jnp.int32)
                    for j in range(nvec):
                        d = cd_v[pl.ds(j * _L, _L)]
                        gi = cg_v[pl.ds(j * _L, _L)]
                        di = plsc.bitcast(jnp.maximum(d, 0.0), jnp.int32)
                        m = (di == t_int) & (gi <= mid)
                        cnt = cnt + plsc.all_reduce_population_count(m)
                    ge = cnt >= needed
                    return (jnp.where(ge, lo, mid + 1), jnp.where(ge, mid, hi))

                nbits = max(1, (nwin * _WIN - 1).bit_length())
                lo0j = jnp.zeros((_L,), jnp.int32)
                hi0j = jnp.full((_L,), nwin * _WIN - 1, jnp.int32)
                _, j_int = lax.fori_loop(0, nbits, jbis_body, (lo0j, hi0j))
                return eq_sum(j_int)

            sv_eq = lax.cond(ceq_sc == need_sc, tie_fast, tie_slow,
                             jnp.int32(0))

            zero_v = jnp.zeros((_L,), jnp.float32)
            wt_v = 1.0 / (t_f + DELTA)
            tot_w = zero_v + jnp.sum(s_w, axis=0)
            tot_wv = zero_v + jnp.sum(s_wv, axis=0)
            sveq_v = zero_v + jnp.sum(sv_eq, axis=0)
            res = ((tot_wv + wt_v * sveq_v)
                   / (tot_w + wt_v * needed.astype(jnp.float32)))
            io = lax.iota(jnp.int32, _L)
            out_v[...] = jnp.where(io == r, res, out_v[...])
            return _carry

        lax.fori_loop(0, rows_per_w, do_row, jnp.int32(0))
        pltpu.sync_copy(out_v, out_hbm.at[pl.ds(base, rows_per_w)])

    return sc_kernel(d2_3d, winmin, tau1d, vals3d, actions)


def kernel(states, actions, W_embed, b_embed, dnd_keys, dnd_values):
    b, d_state = states.shape
    n_actions, mem, d_key = dnd_keys.shape
    mem_pad = _round_up(mem, 8 * _LANE)
    if mem_pad - mem < _WIN:
        mem_pad += 8 * _LANE
    nwin = mem_pad // _WIN

    kt = jnp.transpose(dnd_keys, (0, 2, 1))
    kt = jnp.pad(kt, ((0, 0), (0, 0), (0, mem_pad - mem)))
    vals = jnp.pad(dnd_values, ((0, 0), (0, mem_pad - mem)))
    af = jnp.broadcast_to(actions.astype(jnp.float32)[:, None], (b, _LANE))
    bpad = jnp.zeros((8, _LANE), jnp.float32).at[0, :d_key].set(b_embed)

    d2, winmin, tau = _tc_stage(states, af, W_embed, bpad, kt,
                                n_actions, mem, mem_pad)
    d2_3d = d2.reshape(b, nwin, _WIN)
    vals3d = vals.reshape(n_actions, nwin, _WIN)
    tau1d = tau[:, 0]
    wpc = (mem_pad // 8) // _WIN
    return _sc_stage(d2_3d, winmin, tau1d, vals3d, actions, b, nwin, wpc,
                     TOP_P)


# speedup vs baseline: 36.4578x; 1.3923x over previous
import functools

import jax
import jax.numpy as jnp
from jax import lax
from jax.experimental import pallas as pl
from jax.experimental.pallas import tpu as pltpu
from jax.experimental.pallas import tpu_sc as plsc

TOP_P = 50
DELTA = 1e-3
_LANE = 128
_WIN = 128
_CW_CAP = 128
_EL_CAP = 256
_L = 16


def _round_up(x: int, m: int) -> int:
    return (x + m - 1) // m * m


def _tc_body(nq, nc, cdim, mem, mem_pad, n_actions, top_p, nwin,
             states_ref, actions_ref, w_ref, b_ref, kt_ref,
             d2_ref, winmin_ref, tau_ref, keys_scr, wmin_scr):
    c = pl.program_id(1)
    bq = states_ref.shape[0]
    wpc = cdim // _WIN

    @pl.when(c == 0)
    def _compute_keys():
        st = states_ref[...]
        w = w_ref[...]
        bvec = b_ref[0:1, 0:w.shape[1]]
        keys_scr[...] = jnp.dot(st, w, preferred_element_type=jnp.float32) + bvec
        wmin_scr[...] = jnp.full(wmin_scr.shape, jnp.inf, jnp.float32)

    keys = keys_scr[...]
    qn = jnp.sum(keys * keys, axis=1, keepdims=True)
    af = actions_ref[:, 0:1]

    d2_by_a = []
    for a in range(n_actions):
        kt = kt_ref[a]
        qk = jnp.dot(keys, kt, preferred_element_type=jnp.float32)
        kn = jnp.sum(kt * kt, axis=0, keepdims=True)
        d2_by_a.append(qn - 2.0 * qk + kn)
    if n_actions == 4:
        d2_sel = jnp.where(
            af < 1.5,
            jnp.where(af < 0.5, d2_by_a[0], d2_by_a[1]),
            jnp.where(af < 2.5, d2_by_a[2], d2_by_a[3]))
    else:
        d2_sel = d2_by_a[0]
        for a in range(1, n_actions):
            d2_sel = jnp.where(af == float(a), d2_by_a[a], d2_sel)

    lane_idx = jax.lax.broadcasted_iota(jnp.int32, d2_sel.shape, 1) + c * cdim
    d2_sel = jnp.where(lane_idx >= mem, jnp.inf, d2_sel)
    d2_ref[...] = d2_sel.reshape(d2_ref.shape)

    wmin = jnp.min(d2_sel.reshape(bq, wpc, _WIN), axis=2)
    wpad = jnp.full((bq, _LANE - wpc), jnp.inf, jnp.float32)
    wmin_scr[:, pl.ds(c * _LANE, _LANE)] = jnp.concatenate([wmin, wpad],
                                                           axis=1)

    @pl.when(c == nc - 1)
    def _tau():
        wm = wmin_scr[...]
        wi = jax.lax.bitcast_convert_type(jnp.maximum(wm, 0.0), jnp.int32)
        kth = top_p + 1

        def bis_body(_, lohi):
            lo, hi = lohi
            mid = lo + jax.lax.shift_right_logical(hi - lo, 1)
            cnt = jnp.sum((wi <= mid).astype(jnp.int32), axis=1, keepdims=True)
            ge = cnt >= kth
            return (jnp.where(ge, lo, mid + 1), jnp.where(ge, mid, hi))

        lo0 = jnp.zeros((bq, 1), jnp.int32)
        hi0 = jnp.full((bq, 1), jnp.int32(0x7F800000))
        _, t_int = jax.lax.fori_loop(0, 31, bis_body, (lo0, hi0))
        tau = jax.lax.bitcast_convert_type(t_int, jnp.float32)
        winmin_ref[...] = wm
        tau_ref[...] = jnp.broadcast_to(tau, tau_ref.shape)


def _tc_stage(states, af, W_embed, bpad, kt, n_actions, mem, mem_pad):
    b, d_state = states.shape
    d_key = W_embed.shape[1]
    nq = 2 if b % 256 == 0 else 1
    bq = b // nq
    nc = 7
    assert mem_pad % (nc * 8 * _WIN) == 0, mem_pad
    cdim = mem_pad // nc
    nwin = mem_pad // _WIN
    nwin_pad = nc * _LANE

    body = functools.partial(_tc_body, nq, nc, cdim, mem, mem_pad,
                             n_actions, TOP_P, nwin)
    return pl.pallas_call(
        body,
        grid=(nq, nc),
        in_specs=[
            pl.BlockSpec((bq, d_state), lambda q, c: (q, 0)),
            pl.BlockSpec((bq, _LANE), lambda q, c: (q, 0)),
            pl.BlockSpec((d_state, d_key), lambda q, c: (0, 0)),
            pl.BlockSpec((8, _LANE), lambda q, c: (0, 0)),
            pl.BlockSpec((n_actions, d_key, cdim), lambda q, c: (0, 0, c)),
        ],
        out_specs=[
            pl.BlockSpec((bq, cdim // _WIN, _WIN),
                         lambda q, c: (q, c, 0)),
            pl.BlockSpec((bq, nwin_pad), lambda q, c: (q, 0)),
            pl.BlockSpec((bq, _LANE), lambda q, c: (q, 0)),
        ],
        out_shape=[
            jax.ShapeDtypeStruct((b, nwin, _WIN), jnp.float32),
            jax.ShapeDtypeStruct((b, nwin_pad), jnp.float32),
            jax.ShapeDtypeStruct((b, _LANE), jnp.float32),
        ],
        scratch_shapes=[
            pltpu.VMEM((bq, d_key), jnp.float32),
            pltpu.VMEM((bq, nwin_pad), jnp.float32),
        ],
        compiler_params=pltpu.CompilerParams(
            dimension_semantics=("arbitrary", "arbitrary"),
        ),
    )(states, af, W_embed, bpad, kt)


def _lane_scalar_f32(vec, lane):
    io = lax.iota(jnp.int32, _L)
    return jnp.min(jnp.where(io == lane, vec, jnp.float32(jnp.inf)), axis=0)


def _lane_scalar_i32(vec, lane):
    io = lax.iota(jnp.int32, _L)
    return jnp.max(jnp.where(io == lane, vec, jnp.int32(0)), axis=0)


def _sc_stage(d2_3d, winmin, tau1d, vals3d, actions, b, nwin, wpc, top_p):
    nwin_pad = winmin.shape[1]
    mesh = plsc.VectorSubcoreMesh(core_axis_name="c", subcore_axis_name="s")
    try:
        info = plsc.get_sparse_core_info()
        nc_, ns_ = info.num_cores, info.num_subcores
    except ValueError:
        nc_, ns_ = 2, 16
    nw_workers = nc_ * ns_
    rows_per_w = b // nw_workers
    cap = _CW_CAP
    ecap = _EL_CAP
    nvec = ecap // _L
    pad_win = nwin - 1
    n_chunks = nwin_pad // _L

    @functools.partial(
        pl.kernel,
        out_type=jax.ShapeDtypeStruct((b,), jnp.float32),
        mesh=mesh,
        scratch_types=[
            pltpu.VMEM((nwin_pad,), jnp.float32),
            pltpu.VMEM((cap,), jnp.int32),
            pltpu.VMEM((cap, _WIN), jnp.float32),
            pltpu.VMEM((cap, _WIN), jnp.float32),
            pltpu.VMEM((ecap,), jnp.float32),
            pltpu.VMEM((ecap,), jnp.float32),
            pltpu.VMEM((ecap,), jnp.int32),
            pltpu.VMEM((_L,), jnp.float32),
            pltpu.VMEM((_L,), jnp.int32),
            pltpu.VMEM((_L,), jnp.float32),
            pltpu.SemaphoreType.DMA,
            pltpu.SemaphoreType.DMA,
        ],
        compiler_params=pltpu.CompilerParams(needs_layout_passes=False),
    )
    def sc_kernel(d2_hbm, winmin_hbm, tau_hbm, vals_hbm, act_hbm, out_hbm,
                  wm_v, cw_v, d2w_v, vw_v, cd_v, cv_v, cg_v,
                  tau_v, act_v, out_v, sem1, sem2):
        wid = lax.axis_index("s") * nc_ + lax.axis_index("c")
        base = wid * rows_per_w
        pltpu.sync_copy(tau_hbm.at[pl.ds(base, rows_per_w)], tau_v)
        pltpu.sync_copy(act_hbm.at[pl.ds(base, rows_per_w)], act_v)

        for j in range(cap // _L):
            cw_v[pl.ds(j * _L, _L)] = lax.iota(jnp.int32, _L) + j * _L

        def do_row(r, _carry):
            row = base + r
            pltpu.sync_copy(winmin_hbm.at[row], wm_v)
            tau_r = _lane_scalar_f32(tau_v[...], r)
            a_r = _lane_scalar_i32(act_v[...], r)

            nsub = (wpc + _L - 1) // _L
            cursor = jnp.zeros((_L,), jnp.int32)
            for g in range(n_chunks // (_LANE // _L)):
                for s in range(nsub):
                    wm = wm_v[pl.ds(g * _LANE + s * _L, _L)]
                    msk = wm <= tau_r
                    pos = cursor + jnp.cumsum(msk.astype(jnp.int32)) - 1
                    pos = jnp.minimum(pos, cap - 1)
                    local = lax.iota(jnp.int32, _L) + s * _L
                    idxs = g * wpc + local
                    plsc.store_scatter(cw_v, [pos], idxs, mask=msk)
                    cursor = cursor + plsc.all_reduce_population_count(msk)
            nw_cnt = cursor

            cp1 = pltpu.async_copy(d2_hbm.at[row].at[cw_v], d2w_v, sem1)
            cp2 = pltpu.async_copy(vals_hbm.at[a_r].at[cw_v], vw_v, sem2)
            cp1.wait()
            cp2.wait()

            for j in range(nvec):
                cd_v[pl.ds(j * _L, _L)] = jnp.full((_L,), jnp.inf, jnp.float32)

            nwindows = jnp.max(jnp.minimum(nw_cnt, cap), axis=0)

            def compact_win(j, cursor):
                base16 = (j // _L) * _L
                wchunk = cw_v[pl.ds(base16, _L)]
                lane = j - base16
                wsel = _lane_scalar_i32(wchunk, lane)
                for h in range(_WIN // _L):
                    d = d2w_v[j, pl.ds(h * _L, _L)]
                    v = vw_v[j, pl.ds(h * _L, _L)]
                    msk = d <= tau_r
                    pos = cursor + jnp.cumsum(msk.astype(jnp.int32)) - 1
                    pos = jnp.minimum(pos, ecap - 1)
                    gidx = wsel * _WIN + h * _L + lax.iota(jnp.int32, _L)
                    plsc.store_scatter(cd_v, [pos], d, mask=msk)
                    plsc.store_scatter(cv_v, [pos], v, mask=msk)
                    plsc.store_scatter(cg_v, [pos], gidx, mask=msk)
                    cursor = cursor + plsc.all_reduce_population_count(msk)
                return cursor

            lax.fori_loop(0, nwindows, compact_win,
                          jnp.zeros((_L,), jnp.int32))

            kth = jnp.full((_L,), top_p, jnp.int32)

            def bis_body(_, lohi):
                lo, hi = lohi
                mid = lo + lax.shift_right_logical(hi - lo, 1)
                cnt = jnp.zeros((_L,), jnp.int32)
                for j in range(nvec):
                    d = cd_v[pl.ds(j * _L, _L)]
                    di = plsc.bitcast(jnp.maximum(d, 0.0), jnp.int32)
                    cnt = cnt + plsc.all_reduce_population_count(di <= mid)
                ge = cnt >= kth
                return (jnp.where(ge, lo, mid + 1), jnp.where(ge, mid, hi))

            lo0 = jnp.zeros((_L,), jnp.int32)
            hi0 = jnp.full((_L,), 0x7F800000, jnp.int32)
            _, t_int = lax.fori_loop(0, 31, bis_body, (lo0, hi0))
            t_f = plsc.bitcast(t_int, jnp.float32)

            s_w = jnp.zeros((_L,), jnp.float32)
            s_wv = jnp.zeros((_L,), jnp.float32)
            c_lt = jnp.zeros((_L,), jnp.int32)
            c_eq = jnp.zeros((_L,), jnp.int32)
            for j in range(nvec):
                d = cd_v[pl.ds(j * _L, _L)]
                v = cv_v[pl.ds(j * _L, _L)]
                di = plsc.bitcast(jnp.maximum(d, 0.0), jnp.int32)
                m_lt = di < t_int
                m_eq = di == t_int
                w = 1.0 / (d + DELTA)
                s_w = s_w + jnp.where(m_lt, w, 0.0)
                s_wv = s_wv + jnp.where(m_lt, w * v, 0.0)
                c_lt = c_lt + plsc.all_reduce_population_count(m_lt)
                c_eq = c_eq + plsc.all_reduce_population_count(m_eq)

            needed = jnp.full((_L,), top_p, jnp.int32) - c_lt
            need_sc = _lane_scalar_i32(needed, 0)
            ceq_sc = _lane_scalar_i32(c_eq, 0)

            def eq_sum(limit_v):
                acc = jnp.zeros((_L,), jnp.float32)
                for j in range(nvec):
                    d = cd_v[pl.ds(j * _L, _L)]
                    v = cv_v[pl.ds(j * _L, _L)]
                    gi = cg_v[pl.ds(j * _L, _L)]
                    di = plsc.bitcast(jnp.maximum(d, 0.0), jnp.int32)
                    m = (di == t_int) & (gi <= limit_v)
                    acc = acc + jnp.where(m, v, 0.0)
                return acc

            def tie_fast(_):
                return eq_sum(jnp.full((_L,), 2147483647, jnp.int32))

            def tie_slow(_):
                def jbis_body(_i, lohi):
                    lo, hi = lohi
                    mid = lo + lax.shift_right_logical(hi - lo, 1)
                    cnt = jnp.zeros((_L,), jnp.int32)
                    for j in range(nvec):
                        d = cd_v[pl.ds(j * _L, _L)]
                        gi = cg_v[pl.ds(j * _L, _L)]
                        di = plsc.bitcast(jnp.maximum(d, 0.0), jnp.int32)
                        m = (di == t_int) & (gi <= mid)
                        cnt = cnt + plsc.all_reduce_population_count(m)
                    ge = cnt >= needed
                    return (jnp.where(ge, lo, mid + 1), jnp.where(ge, mid, hi))

                nbits = max(1, (nwin * _WIN - 1).bit_length())
                lo0j = jnp.zeros((_L,), jnp.int32)
                hi0j = jnp.full((_L,), nwin * _WIN - 1, jnp.int32)
                _, j_int = lax.fori_loop(0, nbits, jbis_body, (lo0j, hi0j))
                return eq_sum(j_int)

            sv_eq = lax.cond(ceq_sc == need_sc, tie_fast, tie_slow,
                             jnp.int32(0))

            zero_v = jnp.zeros((_L,), jnp.float32)
            wt_v = 1.0 / (t_f + DELTA)
            tot_w = zero_v + jnp.sum(s_w, axis=0)
            tot_wv = zero_v + jnp.sum(s_wv, axis=0)
            sveq_v = zero_v + jnp.sum(sv_eq, axis=0)
            res = ((tot_wv + wt_v * sveq_v)
                   / (tot_w + wt_v * needed.astype(jnp.float32)))
            io = lax.iota(jnp.int32, _L)
            out_v[...] = jnp.where(io == r, res, out_v[...])
            return _carry

        lax.fori_loop(0, rows_per_w, do_row, jnp.int32(0))
        pltpu.sync_copy(out_v, out_hbm.at[pl.ds(base, rows_per_w)])

    return sc_kernel(d2_3d, winmin, tau1d, vals3d, actions)


def kernel(states, actions, W_embed, b_embed, dnd_keys, dnd_values):
    b, d_state = states.shape
    n_actions, mem, d_key = dnd_keys.shape
    mem_pad = _round_up(mem, 8 * _LANE)
    if mem_pad - mem < _WIN:
        mem_pad += 8 * _LANE
    nwin = mem_pad // _WIN

    kt = jnp.transpose(dnd_keys, (0, 2, 1))
    kt = jnp.pad(kt, ((0, 0), (0, 0), (0, mem_pad - mem)))
    vals = jnp.pad(dnd_values, ((0, 0), (0, mem_pad - mem)))
    af = jnp.broadcast_to(actions.astype(jnp.float32)[:, None], (b, _LANE))
    bpad = jnp.zeros((8, _LANE), jnp.float32).at[0, :d_key].set(b_embed)

    d2_3d, winmin, tau = _tc_stage(states, af, W_embed, bpad, kt,
                                   n_actions, mem, mem_pad)
    vals3d = vals.reshape(n_actions, nwin, _WIN)
    tau1d = tau[:, 0]
    wpc = (mem_pad // 7) // _WIN
    return _sc_stage(d2_3d, winmin, tau1d, vals3d, actions, b, nwin, wpc,
                     TOP_P)
